# baseline (speedup 1.0000x reference)
import functools

import jax
import jax.numpy as jnp
from jax import lax
from jax.experimental import pallas as pl
from jax.experimental.pallas import tpu as pltpu
from jax.experimental.pallas import tpu_sc as plsc

N_ROWS = 100000
D = 64
N_IDX = 16384

NS = 16
L = 16

CHUNK_SIZES = (25024, 25024, 25024, 24928)
CHUNK_STARTS = (0, 25024, 50048, 75072)
MAX_CHUNK = 25024
DUMMY_ROW = MAX_CHUNK
BUF_ROWS = MAX_CHUNK + 8

IDX_PER_TILE = N_IDX // NS
IDX_BLOCKS = IDX_PER_TILE // 128


def _scatter_add_kernel(index_hbm, a_hbm, b_hbm, out_hbm,
                        idx_v, bstage_v, tgt_v, accum_sh):
    c = lax.axis_index("c")
    s = lax.axis_index("s")

    pltpu.sync_copy(index_hbm.at[pl.ds(s * IDX_PER_TILE, IDX_PER_TILE)], idx_v)

    for ci in range(4):
        lo = CHUNK_STARTS[ci]
        n = CHUNK_SIZES[ci]
        rows_per_tile = n // NS

        base = (rows_per_tile // 8) * 8
        rem_granules = (n - base * NS) // 8
        rem_off = base * NS

        @pl.when(c == (ci % 2))
        def _chunk():
            pltpu.sync_copy(
                a_hbm.at[pl.ds(lo + s * base, base), :],
                accum_sh.at[pl.ds(s * base, base), :])

            @pl.when(s < rem_granules)
            def _load_rem():
                pltpu.sync_copy(
                    a_hbm.at[pl.ds(lo + rem_off + s * 8, 8), :],
                    accum_sh.at[pl.ds(rem_off + s * 8, 8), :])

            for j in range(IDX_BLOCKS):
                for v in range(128 // L):
                    iv = idx_v[pl.ds(j * 128 + v * L, L)]
                    m = (iv >= lo) & (iv < lo + n)
                    tgt_v[j, pl.ds(v * L, L)] = jnp.where(m, iv - lo, DUMMY_ROW)

            plsc.subcore_barrier()

            for j in range(IDX_BLOCKS):
                pltpu.sync_copy(
                    b_hbm.at[pl.ds(s * IDX_PER_TILE + j * 128, 128), :],
                    bstage_v)
                pltpu.sync_copy(bstage_v, accum_sh.at[tgt_v.at[j]], add=True)

            plsc.subcore_barrier()

            pltpu.sync_copy(
                accum_sh.at[pl.ds(s * base, base), :],
                out_hbm.at[pl.ds(lo + s * base, base), :])

            @pl.when(s < rem_granules)
            def _store_rem():
                pltpu.sync_copy(
                    accum_sh.at[pl.ds(rem_off + s * 8, 8), :],
                    out_hbm.at[pl.ds(lo + rem_off + s * 8, 8), :])

            plsc.subcore_barrier()


@jax.jit
def _scatter_add(index, a, b):
    run = functools.partial(
        pl.kernel,
        mesh=plsc.VectorSubcoreMesh(core_axis_name="c", subcore_axis_name="s"),
        out_type=jax.ShapeDtypeStruct((N_ROWS, D), jnp.float32),
        scratch_types=[
            pltpu.VMEM((IDX_PER_TILE,), jnp.int32),
            pltpu.VMEM((128, D), jnp.float32),
            pltpu.VMEM((IDX_BLOCKS, 128), jnp.int32),
            pltpu.VMEM_SHARED((BUF_ROWS, D), jnp.float32),
        ],
        compiler_params=pltpu.CompilerParams(use_tc_tiling_on_sc=False),
    )(_scatter_add_kernel)
    return run(index, a, b)


def kernel(index, A, B):
    return _scatter_add(index.astype(jnp.int32), A, B)
